# Initial kernel scaffold; baseline (speedup 1.0000x reference)
import functools

import jax
import jax.numpy as jnp
from jax import lax
from jax.experimental import pallas as pl
from jax.experimental.pallas import tpu as pltpu
from jax.experimental.pallas import tpu_sc as plsc

H = 4096
W = 4096
NBINS = 1024
NC = 2
NS = 16
NW = NC * NS
ROWS_PER_W = H // NW
R = 8
BLOCKS = ROWS_PER_W // R
L = 16
CHUNKS = W // L

_mesh = plsc.VectorSubcoreMesh(core_axis_name="c", subcore_axis_name="s")


@functools.partial(
    pl.kernel,
    out_type=jax.ShapeDtypeStruct((NW, NBINS), jnp.int32),
    mesh=_mesh,
    scratch_types=[
        pltpu.VMEM(((R + 1) * W,), jnp.float32),
        pltpu.VMEM(((R + 2) * W,), jnp.int32),
        pltpu.VMEM((L * NBINS,), jnp.int32),
        pltpu.VMEM((NBINS,), jnp.int32),
    ],
)
def _hist_kernel(img_hbm, out_hbm, fbuf, ibuf, hist, obuf):
    wid = lax.axis_index("s") * NC + lax.axis_index("c")

    iota16 = lax.iota(jnp.int32, L)
    laneoff = iota16 * NBINS
    ones = jnp.ones((L,), jnp.int32)
    mones = -ones
    zeros = jnp.zeros((L,), jnp.int32)

    def zero_hist(i, carry):
        hist[pl.ds(i * L, L)] = zeros
        return carry

    lax.fori_loop(0, (L * NBINS) // L, zero_hist, 0)

    def block(b, carry):
        row0 = wid * ROWS_PER_W + b * R
        pltpu.sync_copy(img_hbm.at[pl.ds(row0 * W, R * W)],
                        fbuf.at[pl.ds(0, R * W)])
        hr = jnp.minimum(row0 + R, H - 1)
        pltpu.sync_copy(img_hbm.at[pl.ds(hr * W, W)],
                        fbuf.at[pl.ds(R * W, W)])

        def conv(t, c):
            off = t * L
            v = fbuf[pl.ds(off, L)]
            y = v * jnp.float32(NBINS - 1)
            ti = y.astype(jnp.int32)
            tf = ti.astype(jnp.float32)
            ibuf[pl.ds(off, L)] = ti + (tf < y).astype(jnp.int32)
            return c

        lax.fori_loop(0, ((R + 1) * W) // L, conv, 0)

        def rowf(r, c0):
            gi = row0 + r
            mrow = jnp.broadcast_to(gi, (L,)) < (H - 1)

            def chunkf(c, c1):
                j = c * L
                base = r * W + j
                ia = ibuf[pl.ds(base, L)]
                iar = ibuf[pl.ds(base + 1, L)]
                iad = ibuf[pl.ds(base + W, L)]
                iadr = ibuf[pl.ds(base + W + 1, L)]
                ix = jnp.maximum(ia, iad)
                iy = jnp.maximum(ia, iar)
                isq = jnp.maximum(iy, jnp.maximum(iad, iadr))
                colv = jnp.broadcast_to(j, (L,)) + iota16
                mcol = colv < (W - 1)
                msq = mcol & mrow
                plsc.addupdate_scatter(hist, [laneoff + ia], ones)
                plsc.addupdate_scatter(hist, [laneoff + ix], mones, mask=mrow)
                plsc.addupdate_scatter(hist, [laneoff + iy], mones, mask=mcol)
                plsc.addupdate_scatter(hist, [laneoff + isq], ones, mask=msq)
                return c1

            lax.fori_loop(0, CHUNKS, chunkf, c0)
            return c0

        lax.fori_loop(0, R, rowf, 0)
        return carry

    lax.fori_loop(0, BLOCKS, block, 0)

    def fold(cidx, carry):
        off = cidx * L
        acc = hist[pl.ds(off, L)]
        for lane in range(1, L):
            acc = acc + hist[pl.ds(lane * NBINS + off, L)]
        obuf[pl.ds(off, L)] = acc
        return carry

    lax.fori_loop(0, NBINS // L, fold, 0)
    pltpu.sync_copy(obuf, out_hbm.at[wid])


@functools.partial(
    pl.kernel,
    out_type=jax.ShapeDtypeStruct((NBINS,), jnp.int32),
    mesh=_mesh,
    scratch_types=[
        pltpu.VMEM((NW, NBINS), jnp.int32),
        pltpu.VMEM((NBINS,), jnp.int32),
    ],
)
def _finalize_kernel(part_hbm, out_hbm, pbuf, obuf):
    wid = lax.axis_index("s") * NC + lax.axis_index("c")

    @pl.when(wid == 0)
    def _():
        pltpu.sync_copy(part_hbm, pbuf)

        def chunk(cidx, carry):
            off = cidx * L
            acc = pbuf[0, pl.ds(off, L)]
            for w in range(1, NW):
                acc = acc + pbuf[w, pl.ds(off, L)]
            cum = plsc.cumsum(acc)
            obuf[pl.ds(off, L)] = cum + jnp.broadcast_to(carry, (L,))
            return carry + jnp.sum(acc)

        lax.fori_loop(0, NBINS // L, chunk, jnp.int32(0))
        pltpu.sync_copy(obuf, out_hbm)


def kernel(img_arr):
    img_flat = img_arr.reshape(-1)
    part = _hist_kernel(img_flat)
    return _finalize_kernel(part)


# speedup vs baseline: 162.5830x; 162.5830x over previous
import functools

import jax
import jax.numpy as jnp
from jax import lax
from jax.experimental import pallas as pl
from jax.experimental.pallas import tpu as pltpu
from jax.experimental.pallas import tpu_sc as plsc

H = 4096
W = 4096
NBINS = 1024
NC = 2
NS = 16
NW = NC * NS
ROWS_PER_W = H // NW
R = 8
BLOCKS = ROWS_PER_W // R
L = 16
CHUNKS = W // L

_mesh = plsc.VectorSubcoreMesh(core_axis_name="c", subcore_axis_name="s")


def _to_bin(v):
    y = v * jnp.float32(NBINS - 1)
    ti = y.astype(jnp.int32)
    return jnp.where(ti.astype(jnp.float32) < y, ti + 1, ti)


@functools.partial(
    pl.kernel,
    out_type=jax.ShapeDtypeStruct((NW, NBINS), jnp.int32),
    mesh=_mesh,
    compiler_params=pltpu.CompilerParams(needs_layout_passes=False),
    scratch_types=[
        pltpu.VMEM(((R + 1) * W,), jnp.float32),
        pltpu.VMEM(((R + 2) * W,), jnp.int32),
        pltpu.VMEM((L * NBINS,), jnp.int32),
        pltpu.VMEM((NBINS,), jnp.int32),
    ],
)
def _hist_kernel(img_hbm, out_hbm, fbuf, ibuf, hist, obuf):
    wid = lax.axis_index("s") * NC + lax.axis_index("c")

    iota16 = lax.iota(jnp.int32, L)
    laneoff = iota16 * NBINS
    ones = jnp.ones((L,), jnp.int32)
    mones = -ones
    zeros = jnp.zeros((L,), jnp.int32)
    mlast = iota16 < (L - 1)

    def zero_hist(i, carry):
        hist[pl.ds(i * L, L)] = zeros
        return carry

    lax.fori_loop(0, (L * NBINS) // L, zero_hist, 0)

    def block(b, carry):
        row0 = wid * ROWS_PER_W + b * R
        pltpu.sync_copy(img_hbm.at[pl.ds(row0 * W, R * W)],
                        fbuf.at[pl.ds(0, R * W)])

        @pl.when(row0 + R < H)
        def _halo():
            pltpu.sync_copy(img_hbm.at[pl.ds((row0 + R) * W, W)],
                            fbuf.at[pl.ds(R * W, W)])

        def conv(t, c):
            off = t * L
            ibuf[pl.ds(off, L)] = _to_bin(fbuf[pl.ds(off, L)])
            return c

        lax.fori_loop(0, ((R + 1) * W) // L, conv, 0)

        def rowf(r, c0):
            gi = row0 + r

            @pl.when(gi < H - 1)
            def _full_row():
                def chunkf(c, c1):
                    base = r * W + c * L
                    ia = ibuf[pl.ds(base, L)]
                    iar = ibuf[pl.ds(base + 1, L)]
                    iad = ibuf[pl.ds(base + W, L)]
                    iadr = ibuf[pl.ds(base + W + 1, L)]
                    ix = jnp.maximum(ia, iad)
                    iy = jnp.maximum(ia, iar)
                    isq = jnp.maximum(iy, jnp.maximum(iad, iadr))
                    plsc.addupdate_scatter(hist, [laneoff + ia], ones)
                    plsc.addupdate_scatter(hist, [laneoff + ix], mones)
                    plsc.addupdate_scatter(hist, [laneoff + iy], mones)
                    plsc.addupdate_scatter(hist, [laneoff + isq], ones)
                    return c1

                lax.fori_loop(0, CHUNKS - 1, chunkf, 0)

                base = r * W + (CHUNKS - 1) * L
                ia = ibuf[pl.ds(base, L)]
                iar = ibuf[pl.ds(base + 1, L)]
                iad = ibuf[pl.ds(base + W, L)]
                iadr = ibuf[pl.ds(base + W + 1, L)]
                ix = jnp.maximum(ia, iad)
                iy = jnp.maximum(ia, iar)
                isq = jnp.maximum(iy, jnp.maximum(iad, iadr))
                plsc.addupdate_scatter(hist, [laneoff + ia], ones)
                plsc.addupdate_scatter(hist, [laneoff + ix], mones)
                plsc.addupdate_scatter(hist, [laneoff + iy], mones, mask=mlast)
                plsc.addupdate_scatter(hist, [laneoff + isq], ones, mask=mlast)

            @pl.when(gi == H - 1)
            def _last_row():
                def chunkv(c, c1):
                    base = r * W + c * L
                    ia = ibuf[pl.ds(base, L)]
                    iar = ibuf[pl.ds(base + 1, L)]
                    iy = jnp.maximum(ia, iar)
                    plsc.addupdate_scatter(hist, [laneoff + ia], ones)
                    plsc.addupdate_scatter(hist, [laneoff + iy], mones)
                    return c1

                lax.fori_loop(0, CHUNKS - 1, chunkv, 0)

                base = r * W + (CHUNKS - 1) * L
                ia = ibuf[pl.ds(base, L)]
                iar = ibuf[pl.ds(base + 1, L)]
                iy = jnp.maximum(ia, iar)
                plsc.addupdate_scatter(hist, [laneoff + ia], ones)
                plsc.addupdate_scatter(hist, [laneoff + iy], mones, mask=mlast)

            return c0

        lax.fori_loop(0, R, rowf, 0)
        return carry

    lax.fori_loop(0, BLOCKS, block, 0)

    def fold(cidx, carry):
        off = cidx * L
        acc = hist[pl.ds(off, L)]
        for lane in range(1, L):
            acc = acc + hist[pl.ds(lane * NBINS + off, L)]
        obuf[pl.ds(off, L)] = acc
        return carry

    lax.fori_loop(0, NBINS // L, fold, 0)
    pltpu.sync_copy(obuf, out_hbm.at[wid])


@functools.partial(
    pl.kernel,
    out_type=jax.ShapeDtypeStruct((NBINS,), jnp.int32),
    mesh=_mesh,
    compiler_params=pltpu.CompilerParams(needs_layout_passes=False),
    scratch_types=[
        pltpu.VMEM((NW, NBINS), jnp.int32),
        pltpu.VMEM((NBINS,), jnp.int32),
    ],
)
def _finalize_kernel(part_hbm, out_hbm, pbuf, obuf):
    wid = lax.axis_index("s") * NC + lax.axis_index("c")

    onehot0 = (lax.iota(jnp.int32, L) == 0).astype(jnp.int32)
    fifteen = jnp.full((L,), L - 1, jnp.int32)

    @pl.when(wid == 0)
    def _():
        pltpu.sync_copy(part_hbm, pbuf)

        def chunk(cidx, carry_vec):
            off = cidx * L
            acc = pbuf[0, pl.ds(off, L)]
            for w in range(1, NW):
                acc = acc + pbuf[w, pl.ds(off, L)]
            acc = acc + carry_vec * onehot0
            cum = plsc.cumsum(acc)
            obuf[pl.ds(off, L)] = cum
            return cum.at[fifteen].get(mode="promise_in_bounds")

        lax.fori_loop(0, NBINS // L, chunk, jnp.zeros((L,), jnp.int32))
        pltpu.sync_copy(obuf, out_hbm)


def kernel(img_arr):
    img_flat = img_arr.reshape(-1)
    part = _hist_kernel(img_flat)
    return _finalize_kernel(part)


# speedup vs baseline: 188.2203x; 1.1577x over previous
import functools

import jax
import jax.numpy as jnp
from jax import lax
from jax.experimental import pallas as pl
from jax.experimental.pallas import tpu as pltpu
from jax.experimental.pallas import tpu_sc as plsc

H = 4096
W = 4096
NBINS = 1024
NC = 2
NS = 16
NW = NC * NS
ROWS_PER_W = H // NW
R = 8
BLOCKS = ROWS_PER_W // R
L = 16
CHUNKS = W // L
BUFLEN = (R + 1) * W + L

_mesh = plsc.VectorSubcoreMesh(core_axis_name="c", subcore_axis_name="s")


def _to_bin(v):
    y = v * jnp.float32(NBINS - 1)
    ti = y.astype(jnp.int32)
    return jnp.where(ti.astype(jnp.float32) < y, ti + 1, ti)


@functools.partial(
    pl.kernel,
    out_type=jax.ShapeDtypeStruct((NW, NBINS), jnp.int32),
    mesh=_mesh,
    compiler_params=pltpu.CompilerParams(needs_layout_passes=False),
    scratch_types=[
        pltpu.VMEM((BUFLEN,), jnp.float32),
        pltpu.VMEM((BUFLEN,), jnp.float32),
        pltpu.VMEM((L * NBINS,), jnp.int32),
        pltpu.VMEM((NBINS,), jnp.int32),
        pltpu.SemaphoreType.DMA,
        pltpu.SemaphoreType.DMA,
    ],
)
def _hist_kernel(img_hbm, out_hbm, fbuf_a, fbuf_b, hist, obuf, sem_a, sem_b):
    wid = lax.axis_index("s") * NC + lax.axis_index("c")

    iota16 = lax.iota(jnp.int32, L)
    laneoff = iota16 * NBINS
    ones = jnp.ones((L,), jnp.int32)
    mones = -ones
    zeros = jnp.zeros((L,), jnp.int32)
    mlast = iota16 < (L - 1)

    def zero_hist(i, carry):
        hist[pl.ds(i * L, L)] = zeros
        return carry

    lax.fori_loop(0, (L * NBINS) // L, zero_hist, 0)

    def dma_pair(bb, buf, sem):
        row0 = wid * ROWS_PER_W + bb * R
        main = pltpu.make_async_copy(img_hbm.at[pl.ds(row0 * W, R * W)],
                                     buf.at[pl.ds(0, R * W)], sem)
        halo = pltpu.make_async_copy(img_hbm.at[pl.ds((row0 + R) * W, W)],
                                     buf.at[pl.ds(R * W, W)], sem)
        return main, halo, row0 + R < H

    def start_dma(bb, buf, sem):
        main, halo, has_halo = dma_pair(bb, buf, sem)
        main.start()

        @pl.when(has_halo)
        def _():
            halo.start()

    def wait_dma(bb, buf, sem):
        main, halo, has_halo = dma_pair(bb, buf, sem)
        main.wait()

        @pl.when(has_halo)
        def _():
            halo.wait()

    def process(bb, buf):
        row0 = wid * ROWS_PER_W + bb * R

        def conv(t, c):
            off = t * L
            idx = _to_bin(buf[pl.ds(off, L)])
            buf[pl.ds(off, L)] = plsc.bitcast(idx, jnp.float32)
            return c

        lax.fori_loop(0, ((R + 1) * W) // L, conv, 0, unroll=4)

        def bins(off):
            return plsc.bitcast(buf[pl.ds(off, L)], jnp.int32)

        def rowf(r, c0):
            gi = row0 + r

            @pl.when(gi < H - 1)
            def _full_row():
                def chunkf(c, c1):
                    base = r * W + c * L
                    ia = bins(base)
                    iar = bins(base + 1)
                    iad = bins(base + W)
                    iadr = bins(base + W + 1)
                    ix = jnp.maximum(ia, iad)
                    iy = jnp.maximum(ia, iar)
                    isq = jnp.maximum(iy, jnp.maximum(iad, iadr))
                    plsc.addupdate_scatter(hist, [laneoff + ia], ones)
                    plsc.addupdate_scatter(hist, [laneoff + ix], mones)
                    plsc.addupdate_scatter(hist, [laneoff + iy], mones)
                    plsc.addupdate_scatter(hist, [laneoff + isq], ones)
                    return c1

                lax.fori_loop(0, CHUNKS - 1, chunkf, 0, unroll=5)

                base = r * W + (CHUNKS - 1) * L
                ia = bins(base)
                iar = bins(base + 1)
                iad = bins(base + W)
                iadr = bins(base + W + 1)
                ix = jnp.maximum(ia, iad)
                iy = jnp.maximum(ia, iar)
                isq = jnp.maximum(iy, jnp.maximum(iad, iadr))
                plsc.addupdate_scatter(hist, [laneoff + ia], ones)
                plsc.addupdate_scatter(hist, [laneoff + ix], mones)
                plsc.addupdate_scatter(hist, [laneoff + iy], mones, mask=mlast)
                plsc.addupdate_scatter(hist, [laneoff + isq], ones, mask=mlast)

            @pl.when(gi == H - 1)
            def _last_row():
                def chunkv(c, c1):
                    base = r * W + c * L
                    ia = bins(base)
                    iar = bins(base + 1)
                    iy = jnp.maximum(ia, iar)
                    plsc.addupdate_scatter(hist, [laneoff + ia], ones)
                    plsc.addupdate_scatter(hist, [laneoff + iy], mones)
                    return c1

                lax.fori_loop(0, CHUNKS - 1, chunkv, 0, unroll=5)

                base = r * W + (CHUNKS - 1) * L
                ia = bins(base)
                iar = bins(base + 1)
                iy = jnp.maximum(ia, iar)
                plsc.addupdate_scatter(hist, [laneoff + ia], ones)
                plsc.addupdate_scatter(hist, [laneoff + iy], mones, mask=mlast)

            return c0

        lax.fori_loop(0, R, rowf, 0)

    start_dma(0, fbuf_a, sem_a)

    def outer(k, carry):
        b0 = 2 * k
        wait_dma(b0, fbuf_a, sem_a)
        start_dma(b0 + 1, fbuf_b, sem_b)
        process(b0, fbuf_a)
        wait_dma(b0 + 1, fbuf_b, sem_b)

        @pl.when(b0 + 2 < BLOCKS)
        def _():
            start_dma(b0 + 2, fbuf_a, sem_a)

        process(b0 + 1, fbuf_b)
        return carry

    lax.fori_loop(0, BLOCKS // 2, outer, 0)

    def fold(cidx, carry):
        off = cidx * L
        acc = hist[pl.ds(off, L)]
        for lane in range(1, L):
            acc = acc + hist[pl.ds(lane * NBINS + off, L)]
        obuf[pl.ds(off, L)] = acc
        return carry

    lax.fori_loop(0, NBINS // L, fold, 0)
    pltpu.sync_copy(obuf, out_hbm.at[wid])


@functools.partial(
    pl.kernel,
    out_type=jax.ShapeDtypeStruct((NBINS,), jnp.int32),
    mesh=_mesh,
    compiler_params=pltpu.CompilerParams(needs_layout_passes=False),
    scratch_types=[
        pltpu.VMEM((NW, NBINS), jnp.int32),
        pltpu.VMEM((NBINS,), jnp.int32),
    ],
)
def _finalize_kernel(part_hbm, out_hbm, pbuf, obuf):
    wid = lax.axis_index("s") * NC + lax.axis_index("c")

    onehot0 = (lax.iota(jnp.int32, L) == 0).astype(jnp.int32)
    fifteen = jnp.full((L,), L - 1, jnp.int32)

    @pl.when(wid == 0)
    def _():
        pltpu.sync_copy(part_hbm, pbuf)

        def chunk(cidx, carry_vec):
            off = cidx * L
            acc = pbuf[0, pl.ds(off, L)]
            for w in range(1, NW):
                acc = acc + pbuf[w, pl.ds(off, L)]
            acc = acc + carry_vec * onehot0
            cum = plsc.cumsum(acc)
            obuf[pl.ds(off, L)] = cum
            return cum.at[fifteen].get(mode="promise_in_bounds")

        lax.fori_loop(0, NBINS // L, chunk, jnp.zeros((L,), jnp.int32))
        pltpu.sync_copy(obuf, out_hbm)


def kernel(img_arr):
    img_flat = img_arr.reshape(-1)
    part = _hist_kernel(img_flat)
    return _finalize_kernel(part)


# speedup vs baseline: 219.3701x; 1.1655x over previous
import functools

import jax
import jax.numpy as jnp
from jax import lax
from jax.experimental import pallas as pl
from jax.experimental.pallas import tpu as pltpu
from jax.experimental.pallas import tpu_sc as plsc

H = 4096
W = 4096
NBINS = 1024
NC = 2
NS = 16
NW = NC * NS
ROWS_PER_W = H // NW
R = 8
BLOCKS = ROWS_PER_W // R
L = 16
CHUNKS = W // L
BUFLEN = (R + 1) * W + L

_mesh = plsc.VectorSubcoreMesh(core_axis_name="c", subcore_axis_name="s")


def _to_bin(v):
    y = v * jnp.float32(NBINS - 1)
    ti = y.astype(jnp.int32)
    return jnp.where(ti.astype(jnp.float32) < y, ti + 1, ti)


@functools.partial(
    pl.kernel,
    out_type=jax.ShapeDtypeStruct((NW, NBINS), jnp.int32),
    mesh=_mesh,
    compiler_params=pltpu.CompilerParams(needs_layout_passes=False),
    scratch_types=[
        pltpu.VMEM((BUFLEN,), jnp.float32),
        pltpu.VMEM((BUFLEN,), jnp.float32),
        pltpu.VMEM((L * NBINS,), jnp.int32),
        pltpu.VMEM((NBINS,), jnp.int32),
        pltpu.SemaphoreType.DMA,
        pltpu.SemaphoreType.DMA,
    ],
)
def _hist_kernel(img_hbm, out_hbm, fbuf_a, fbuf_b, hist, obuf, sem_a, sem_b):
    wid = lax.axis_index("s") * NC + lax.axis_index("c")

    iota16 = lax.iota(jnp.int32, L)
    laneoff = iota16 * NBINS
    ones = jnp.ones((L,), jnp.int32)
    mones = -ones
    zeros = jnp.zeros((L,), jnp.int32)
    mlast = iota16 < (L - 1)

    def zero_hist(i, carry):
        hist[pl.ds(i * L, L)] = zeros
        return carry

    lax.fori_loop(0, (L * NBINS) // L, zero_hist, 0)

    def dma_pair(bb, buf, sem):
        row0 = wid * ROWS_PER_W + bb * R
        main = pltpu.make_async_copy(img_hbm.at[pl.ds(row0 * W, R * W)],
                                     buf.at[pl.ds(0, R * W)], sem)
        halo = pltpu.make_async_copy(img_hbm.at[pl.ds((row0 + R) * W, W)],
                                     buf.at[pl.ds(R * W, W)], sem)
        return main, halo, row0 + R < H

    def start_dma(bb, buf, sem):
        main, halo, has_halo = dma_pair(bb, buf, sem)
        main.start()

        @pl.when(has_halo)
        def _():
            halo.start()

    def wait_dma(bb, buf, sem):
        main, halo, has_halo = dma_pair(bb, buf, sem)
        main.wait()

        @pl.when(has_halo)
        def _():
            halo.wait()

    def process(bb, buf):
        row0 = wid * ROWS_PER_W + bb * R

        def conv(t, c):
            off = t * L
            idx = (_to_bin(buf[pl.ds(off, L)]) << 4) | iota16
            buf[pl.ds(off, L)] = plsc.bitcast(idx, jnp.float32)
            return c

        lax.fori_loop(0, ((R + 1) * W) // L, conv, 0, unroll=4)

        def bins(off):
            return plsc.bitcast(buf[pl.ds(off, L)], jnp.int32)

        def rowf(r, c0):
            gi = row0 + r

            @pl.when(gi < H - 1)
            def _full_row():
                def chunkf(c, c1):
                    base = r * W + c * L
                    ia = bins(base)
                    iar = bins(base + 1)
                    iad = bins(base + W)
                    iadr = bins(base + W + 1)
                    ix = jnp.maximum(ia, iad)
                    iy = jnp.maximum(ia, iar)
                    isq = jnp.maximum(iy, jnp.maximum(iad, iadr))
                    iy = (iy & -16) | iota16
                    isq = (isq & -16) | iota16
                    plsc.addupdate_scatter(hist, [ia], ones)
                    plsc.addupdate_scatter(hist, [ix], mones)
                    plsc.addupdate_scatter(hist, [iy], mones)
                    plsc.addupdate_scatter(hist, [isq], ones)
                    return c1

                lax.fori_loop(0, CHUNKS - 1, chunkf, 0, unroll=5)

                base = r * W + (CHUNKS - 1) * L
                ia = bins(base)
                iar = bins(base + 1)
                iad = bins(base + W)
                iadr = bins(base + W + 1)
                ix = jnp.maximum(ia, iad)
                iy = jnp.maximum(ia, iar)
                isq = jnp.maximum(iy, jnp.maximum(iad, iadr))
                iy = (iy & -16) | iota16
                isq = (isq & -16) | iota16
                plsc.addupdate_scatter(hist, [ia], ones)
                plsc.addupdate_scatter(hist, [ix], mones)
                plsc.addupdate_scatter(hist, [iy], mones, mask=mlast)
                plsc.addupdate_scatter(hist, [isq], ones, mask=mlast)

            @pl.when(gi == H - 1)
            def _last_row():
                def chunkv(c, c1):
                    base = r * W + c * L
                    ia = bins(base)
                    iar = bins(base + 1)
                    iy = (jnp.maximum(ia, iar) & -16) | iota16
                    plsc.addupdate_scatter(hist, [ia], ones)
                    plsc.addupdate_scatter(hist, [iy], mones)
                    return c1

                lax.fori_loop(0, CHUNKS - 1, chunkv, 0, unroll=5)

                base = r * W + (CHUNKS - 1) * L
                ia = bins(base)
                iar = bins(base + 1)
                iy = (jnp.maximum(ia, iar) & -16) | iota16
                plsc.addupdate_scatter(hist, [ia], ones)
                plsc.addupdate_scatter(hist, [iy], mones, mask=mlast)

            return c0

        lax.fori_loop(0, R, rowf, 0)

    start_dma(0, fbuf_a, sem_a)

    def outer(k, carry):
        b0 = 2 * k
        wait_dma(b0, fbuf_a, sem_a)
        start_dma(b0 + 1, fbuf_b, sem_b)
        process(b0, fbuf_a)
        wait_dma(b0 + 1, fbuf_b, sem_b)

        @pl.when(b0 + 2 < BLOCKS)
        def _():
            start_dma(b0 + 2, fbuf_a, sem_a)

        process(b0 + 1, fbuf_b)
        return carry

    lax.fori_loop(0, BLOCKS // 2, outer, 0)

    gidx = iota16 * L

    def fold(cidx, carry):
        acc = plsc.load_gather(hist, [gidx + cidx * (L * L)])
        for lane in range(1, L):
            acc = acc + plsc.load_gather(hist, [gidx + (cidx * (L * L) + lane)])
        obuf[pl.ds(cidx * L, L)] = acc
        return carry

    lax.fori_loop(0, NBINS // L, fold, 0)
    pltpu.sync_copy(obuf, out_hbm.at[wid])


@functools.partial(
    pl.kernel,
    out_type=jax.ShapeDtypeStruct((NBINS,), jnp.int32),
    mesh=_mesh,
    compiler_params=pltpu.CompilerParams(needs_layout_passes=False),
    scratch_types=[
        pltpu.VMEM((NW, NBINS), jnp.int32),
        pltpu.VMEM((NBINS,), jnp.int32),
    ],
)
def _finalize_kernel(part_hbm, out_hbm, pbuf, obuf):
    wid = lax.axis_index("s") * NC + lax.axis_index("c")

    onehot0 = (lax.iota(jnp.int32, L) == 0).astype(jnp.int32)
    fifteen = jnp.full((L,), L - 1, jnp.int32)

    @pl.when(wid == 0)
    def _():
        pltpu.sync_copy(part_hbm, pbuf)

        def chunk(cidx, carry_vec):
            off = cidx * L
            acc = pbuf[0, pl.ds(off, L)]
            for w in range(1, NW):
                acc = acc + pbuf[w, pl.ds(off, L)]
            acc = acc + carry_vec * onehot0
            cum = plsc.cumsum(acc)
            obuf[pl.ds(off, L)] = cum
            return cum.at[fifteen].get(mode="promise_in_bounds")

        lax.fori_loop(0, NBINS // L, chunk, jnp.zeros((L,), jnp.int32))
        pltpu.sync_copy(obuf, out_hbm)


def kernel(img_arr):
    img_flat = img_arr.reshape(-1)
    part = _hist_kernel(img_flat)
    return _finalize_kernel(part)


# speedup vs baseline: 382.4130x; 1.7432x over previous
import functools

import jax
import jax.numpy as jnp
from jax import lax
from jax.experimental import pallas as pl
from jax.experimental.pallas import tpu as pltpu
from jax.experimental.pallas import tpu_sc as plsc

H = 4096
W = 4096
NBINS = 1024
NC = 2
NS = 16
NW = NC * NS
ROWS_PER_W = H // NW
R = 8
BLOCKS = ROWS_PER_W // R
L = 16
CHUNKS = W // L
BUFLEN = (R + 1) * W + L

_mesh = plsc.VectorSubcoreMesh(core_axis_name="c", subcore_axis_name="s")


def _to_bin(v):
    y = v * jnp.float32(NBINS - 1)
    ti = y.astype(jnp.int32)
    return jnp.where(ti.astype(jnp.float32) < y, ti + 1, ti)


@functools.partial(
    pl.kernel,
    out_type=jax.ShapeDtypeStruct((NW, NBINS), jnp.int32),
    mesh=_mesh,
    compiler_params=pltpu.CompilerParams(needs_layout_passes=False),
    scratch_types=[
        pltpu.VMEM((BUFLEN,), jnp.float32),
        pltpu.VMEM((BUFLEN,), jnp.float32),
        pltpu.VMEM((L * NBINS,), jnp.int32),
        pltpu.VMEM((NBINS,), jnp.int32),
        pltpu.SemaphoreType.DMA,
        pltpu.SemaphoreType.DMA,
    ],
)
def _hist_kernel(img_hbm, out_hbm, fbuf_a, fbuf_b, hist, obuf, sem_a, sem_b):
    wid = lax.axis_index("s") * NC + lax.axis_index("c")

    iota16 = lax.iota(jnp.int32, L)
    iota_u = lax.iota(jnp.uint32, L)
    ones = jnp.ones((L,), jnp.int32)
    mones = -ones
    zeros = jnp.zeros((L,), jnp.int32)
    mlast = iota16 < (L - 1)

    def zero_hist(i, carry):
        hist[pl.ds(i * L, L)] = zeros
        return carry

    lax.fori_loop(0, (L * NBINS) // L, zero_hist, 0)

    def dma_pair(bb, buf, sem):
        row0 = wid * ROWS_PER_W + bb * R
        main = pltpu.make_async_copy(img_hbm.at[pl.ds(row0 * W, R * W)],
                                     buf.at[pl.ds(0, R * W)], sem)
        halo = pltpu.make_async_copy(img_hbm.at[pl.ds((row0 + R) * W, W)],
                                     buf.at[pl.ds(R * W, W)], sem)
        return main, halo, row0 + R < H

    def start_dma(bb, buf, sem):
        main, halo, has_halo = dma_pair(bb, buf, sem)
        main.start()

        @pl.when(has_halo)
        def _():
            halo.start()

    def wait_dma(bb, buf, sem):
        main, halo, has_halo = dma_pair(bb, buf, sem)
        main.wait()

        @pl.when(has_halo)
        def _():
            halo.wait()

    def process(bb, buf):
        row0 = wid * ROWS_PER_W + bb * R

        @plsc.parallel_loop(0, ((R + 1) * W) // L, unroll=4)
        def conv(t):
            off = t * L
            idx = (_to_bin(buf[pl.ds(off, L)]) << 4) | iota16
            buf[pl.ds(off, L)] = plsc.bitcast(idx, jnp.float32)

        def bins(off):
            return plsc.bitcast(buf[pl.ds(off, L)], jnp.uint32)

        def scat(idx_u, val, mask=None):
            plsc.addupdate_scatter(hist, [plsc.bitcast(idx_u, jnp.int32)],
                                   val, mask=mask)

        def retag(idx_u):
            return (idx_u & jnp.uint32(0xFFFFFFF0)) | iota_u

        def rowf(r, c0):
            gi = row0 + r

            @pl.when(gi < H - 1)
            def _full_row():
                @plsc.parallel_loop(0, CHUNKS - 1, unroll=5)
                def chunkf(c):
                    base = r * W + c * L
                    ia = bins(base)
                    iar = bins(base + 1)
                    iad = bins(base + W)
                    iadr = bins(base + W + 1)
                    ix = jnp.maximum(ia, iad)
                    iy = jnp.maximum(ia, iar)
                    isq = jnp.maximum(iy, jnp.maximum(iad, iadr))
                    scat(ia, ones)
                    scat(ix, mones)
                    scat(retag(iy), mones)
                    scat(retag(isq), ones)

                base = r * W + (CHUNKS - 1) * L
                ia = bins(base)
                iar = bins(base + 1)
                iad = bins(base + W)
                iadr = bins(base + W + 1)
                ix = jnp.maximum(ia, iad)
                iy = jnp.maximum(ia, iar)
                isq = jnp.maximum(iy, jnp.maximum(iad, iadr))
                scat(ia, ones)
                scat(ix, mones)
                scat(retag(iy), mones, mask=mlast)
                scat(retag(isq), ones, mask=mlast)

            @pl.when(gi == H - 1)
            def _last_row():
                @plsc.parallel_loop(0, CHUNKS - 1, unroll=5)
                def chunkv(c):
                    base = r * W + c * L
                    ia = bins(base)
                    iar = bins(base + 1)
                    scat(ia, ones)
                    scat(retag(jnp.maximum(ia, iar)), mones)

                base = r * W + (CHUNKS - 1) * L
                ia = bins(base)
                iar = bins(base + 1)
                scat(ia, ones)
                scat(retag(jnp.maximum(ia, iar)), mones, mask=mlast)

            return c0

        lax.fori_loop(0, R, rowf, 0)

    start_dma(0, fbuf_a, sem_a)

    def outer(k, carry):
        b0 = 2 * k
        wait_dma(b0, fbuf_a, sem_a)
        start_dma(b0 + 1, fbuf_b, sem_b)
        process(b0, fbuf_a)
        wait_dma(b0 + 1, fbuf_b, sem_b)

        @pl.when(b0 + 2 < BLOCKS)
        def _():
            start_dma(b0 + 2, fbuf_a, sem_a)

        process(b0 + 1, fbuf_b)
        return carry

    lax.fori_loop(0, BLOCKS // 2, outer, 0)

    gidx = iota16 * L

    def fold(cidx, carry):
        acc = plsc.load_gather(hist, [gidx + cidx * (L * L)])
        for lane in range(1, L):
            acc = acc + plsc.load_gather(hist, [gidx + (cidx * (L * L) + lane)])
        obuf[pl.ds(cidx * L, L)] = acc
        return carry

    lax.fori_loop(0, NBINS // L, fold, 0)
    pltpu.sync_copy(obuf, out_hbm.at[wid])


@functools.partial(
    pl.kernel,
    out_type=jax.ShapeDtypeStruct((NBINS,), jnp.int32),
    mesh=_mesh,
    compiler_params=pltpu.CompilerParams(needs_layout_passes=False),
    scratch_types=[
        pltpu.VMEM((NW, NBINS), jnp.int32),
        pltpu.VMEM((NBINS,), jnp.int32),
    ],
)
def _finalize_kernel(part_hbm, out_hbm, pbuf, obuf):
    wid = lax.axis_index("s") * NC + lax.axis_index("c")

    onehot0 = (lax.iota(jnp.int32, L) == 0).astype(jnp.int32)
    fifteen = jnp.full((L,), L - 1, jnp.int32)

    @pl.when(wid == 0)
    def _():
        pltpu.sync_copy(part_hbm, pbuf)

        def chunk(cidx, carry_vec):
            off = cidx * L
            acc = pbuf[0, pl.ds(off, L)]
            for w in range(1, NW):
                acc = acc + pbuf[w, pl.ds(off, L)]
            acc = acc + carry_vec * onehot0
            cum = plsc.cumsum(acc)
            obuf[pl.ds(off, L)] = cum
            return cum.at[fifteen].get(mode="promise_in_bounds")

        lax.fori_loop(0, NBINS // L, chunk, jnp.zeros((L,), jnp.int32))
        pltpu.sync_copy(obuf, out_hbm)


def kernel(img_arr):
    img_flat = img_arr.reshape(-1)
    part = _hist_kernel(img_flat)
    return _finalize_kernel(part)


# speedup vs baseline: 455.3188x; 1.1906x over previous
import functools

import jax
import jax.numpy as jnp
from jax import lax
from jax.experimental import pallas as pl
from jax.experimental.pallas import tpu as pltpu
from jax.experimental.pallas import tpu_sc as plsc

H = 4096
W = 4096
NBINS = 1024
NC = 2
NS = 16
NW = NC * NS
ROWS_PER_W = H // NW
R = 8
BLOCKS = ROWS_PER_W // R
L = 16
CHUNKS = W // L
BUFLEN = (R + 1) * W + L

_mesh = plsc.VectorSubcoreMesh(core_axis_name="c", subcore_axis_name="s")


def _to_bin(v):
    y = v * jnp.float32(NBINS - 1)
    ti = y.astype(jnp.int32)
    return jnp.where(ti.astype(jnp.float32) < y, ti + 1, ti)


@functools.partial(
    pl.kernel,
    out_type=jax.ShapeDtypeStruct((NW, NBINS), jnp.int32),
    mesh=_mesh,
    compiler_params=pltpu.CompilerParams(needs_layout_passes=False),
    scratch_types=[
        pltpu.VMEM((BUFLEN,), jnp.float32),
        pltpu.VMEM((BUFLEN,), jnp.float32),
        pltpu.VMEM((L * NBINS,), jnp.int32),
        pltpu.VMEM((NBINS,), jnp.int32),
        pltpu.SemaphoreType.DMA,
        pltpu.SemaphoreType.DMA,
    ],
)
def _hist_kernel(img_hbm, out_hbm, fbuf_a, fbuf_b, hist, obuf, sem_a, sem_b):
    wid = lax.axis_index("s") * NC + lax.axis_index("c")

    iota16 = lax.iota(jnp.int32, L)
    iota_u = lax.iota(jnp.uint32, L)
    ones = jnp.ones((L,), jnp.int32)
    mones = -ones
    zeros = jnp.zeros((L,), jnp.int32)
    mlast = iota16 < (L - 1)

    def zero_hist(i, carry):
        hist[pl.ds(i * L, L)] = zeros
        return carry

    lax.fori_loop(0, (L * NBINS) // L, zero_hist, 0)

    def dma_pair(bb, buf, sem):
        row0 = wid * ROWS_PER_W + bb * R
        main = pltpu.make_async_copy(img_hbm.at[pl.ds(row0 * W, R * W)],
                                     buf.at[pl.ds(0, R * W)], sem)
        halo = pltpu.make_async_copy(img_hbm.at[pl.ds((row0 + R) * W, W)],
                                     buf.at[pl.ds(R * W, W)], sem)
        return main, halo, row0 + R < H

    def start_dma(bb, buf, sem):
        main, halo, has_halo = dma_pair(bb, buf, sem)
        main.start()

        @pl.when(has_halo)
        def _():
            halo.start()

    def wait_dma(bb, buf, sem):
        main, halo, has_halo = dma_pair(bb, buf, sem)
        main.wait()

        @pl.when(has_halo)
        def _():
            halo.wait()

    def process(bb, buf):
        row0 = wid * ROWS_PER_W + bb * R
        fast = row0 + R < H

        def conv_chunk(off):
            b = _to_bin(buf[pl.ds(off, L)])
            return plsc.bitcast((b << 4) | iota16, jnp.uint32)

        def scat(idx_u, val, mask=None):
            plsc.addupdate_scatter(hist, [plsc.bitcast(idx_u, jnp.int32)],
                                   val, mask=mask)

        @pl.when(fast)
        def _fast():
            def column(jbase, edge_mask):
                ia = conv_chunk(jbase)
                iar = conv_chunk(jbase + 1)
                for r in range(R):
                    down = (r + 1) * W
                    iad = conv_chunk(jbase + down)
                    iadr = conv_chunk(jbase + down + 1)
                    ix = jnp.maximum(ia, iad)
                    iy = jnp.maximum(ia, iar)
                    isq = jnp.maximum(iy, jnp.maximum(iad, iadr))
                    scat(ia, ones)
                    scat(ix, mones)
                    scat(iy, mones, mask=edge_mask)
                    scat(isq, ones, mask=edge_mask)
                    ia, iar = iad, iadr

            @plsc.parallel_loop(0, CHUNKS - 1)
            def chunkf(c):
                column(c * L, None)

            column((CHUNKS - 1) * L, mlast)

        @pl.when(jnp.logical_not(fast))
        def _slow():
            @plsc.parallel_loop(0, (R * W) // L, unroll=4)
            def conv(t):
                off = t * L
                buf[pl.ds(off, L)] = plsc.bitcast(conv_chunk(off),
                                                  jnp.float32)

            def bins(off):
                return plsc.bitcast(buf[pl.ds(off, L)], jnp.uint32)

            def retag(idx_u):
                return (idx_u & jnp.uint32(0xFFFFFFF0)) | iota_u

            def rowf(r, c0):
                gi = row0 + r

                @pl.when(gi < H - 1)
                def _full_row():
                    @plsc.parallel_loop(0, CHUNKS - 1, unroll=5)
                    def chunkf(c):
                        base = r * W + c * L
                        ia = bins(base)
                        iar = bins(base + 1)
                        iad = bins(base + W)
                        iadr = bins(base + W + 1)
                        ix = jnp.maximum(ia, iad)
                        iy = jnp.maximum(ia, iar)
                        isq = jnp.maximum(iy, jnp.maximum(iad, iadr))
                        scat(ia, ones)
                        scat(ix, mones)
                        scat(retag(iy), mones)
                        scat(retag(isq), ones)

                    base = r * W + (CHUNKS - 1) * L
                    ia = bins(base)
                    iar = bins(base + 1)
                    iad = bins(base + W)
                    iadr = bins(base + W + 1)
                    ix = jnp.maximum(ia, iad)
                    iy = jnp.maximum(ia, iar)
                    isq = jnp.maximum(iy, jnp.maximum(iad, iadr))
                    scat(ia, ones)
                    scat(ix, mones)
                    scat(retag(iy), mones, mask=mlast)
                    scat(retag(isq), ones, mask=mlast)

                @pl.when(gi == H - 1)
                def _last_row():
                    @plsc.parallel_loop(0, CHUNKS - 1, unroll=5)
                    def chunkv(c):
                        base = r * W + c * L
                        ia = bins(base)
                        iar = bins(base + 1)
                        scat(ia, ones)
                        scat(retag(jnp.maximum(ia, iar)), mones)

                    base = r * W + (CHUNKS - 1) * L
                    ia = bins(base)
                    iar = bins(base + 1)
                    scat(ia, ones)
                    scat(retag(jnp.maximum(ia, iar)), mones, mask=mlast)

                return c0

            lax.fori_loop(0, R, rowf, 0)

    start_dma(0, fbuf_a, sem_a)

    def outer(k, carry):
        b0 = 2 * k
        wait_dma(b0, fbuf_a, sem_a)
        start_dma(b0 + 1, fbuf_b, sem_b)
        process(b0, fbuf_a)
        wait_dma(b0 + 1, fbuf_b, sem_b)

        @pl.when(b0 + 2 < BLOCKS)
        def _():
            start_dma(b0 + 2, fbuf_a, sem_a)

        process(b0 + 1, fbuf_b)
        return carry

    lax.fori_loop(0, BLOCKS // 2, outer, 0)

    gidx = iota16 * L

    def fold(cidx, carry):
        acc = plsc.load_gather(hist, [gidx + cidx * (L * L)])
        for lane in range(1, L):
            acc = acc + plsc.load_gather(hist, [gidx + (cidx * (L * L) + lane)])
        obuf[pl.ds(cidx * L, L)] = acc
        return carry

    lax.fori_loop(0, NBINS // L, fold, 0)
    pltpu.sync_copy(obuf, out_hbm.at[wid])


@functools.partial(
    pl.kernel,
    out_type=jax.ShapeDtypeStruct((NBINS,), jnp.int32),
    mesh=_mesh,
    compiler_params=pltpu.CompilerParams(needs_layout_passes=False),
    scratch_types=[
        pltpu.VMEM((NW, NBINS), jnp.int32),
        pltpu.VMEM((NBINS,), jnp.int32),
    ],
)
def _finalize_kernel(part_hbm, out_hbm, pbuf, obuf):
    wid = lax.axis_index("s") * NC + lax.axis_index("c")

    onehot0 = (lax.iota(jnp.int32, L) == 0).astype(jnp.int32)
    fifteen = jnp.full((L,), L - 1, jnp.int32)

    @pl.when(wid == 0)
    def _():
        pltpu.sync_copy(part_hbm, pbuf)

        def chunk(cidx, carry_vec):
            off = cidx * L
            acc = pbuf[0, pl.ds(off, L)]
            for w in range(1, NW):
                acc = acc + pbuf[w, pl.ds(off, L)]
            acc = acc + carry_vec * onehot0
            cum = plsc.cumsum(acc)
            obuf[pl.ds(off, L)] = cum
            return cum.at[fifteen].get(mode="promise_in_bounds")

        lax.fori_loop(0, NBINS // L, chunk, jnp.zeros((L,), jnp.int32))
        pltpu.sync_copy(obuf, out_hbm)


def kernel(img_arr):
    img_flat = img_arr.reshape(-1)
    part = _hist_kernel(img_flat)
    return _finalize_kernel(part)


# speedup vs baseline: 562.2744x; 1.2349x over previous
import functools

import jax
import jax.numpy as jnp
from jax import lax
from jax.experimental import pallas as pl
from jax.experimental.pallas import tpu as pltpu
from jax.experimental.pallas import tpu_sc as plsc

H = 4096
W = 4096
NBINS = 1024
NC = 2
NS = 16
NW = NC * NS
ROWS_PER_W = H // NW
R = 8
BLOCKS = ROWS_PER_W // R
L = 16
CHUNKS = W // L
BUFLEN = (R + 1) * W + L

_mesh = plsc.VectorSubcoreMesh(core_axis_name="c", subcore_axis_name="s")


def _to_bin(v):
    y = v * jnp.float32(NBINS - 1)
    ti = y.astype(jnp.int32)
    return jnp.where(ti.astype(jnp.float32) < y, ti + 1, ti)


@functools.partial(
    pl.kernel,
    out_type=jax.ShapeDtypeStruct((NW, NBINS), jnp.int32),
    mesh=_mesh,
    compiler_params=pltpu.CompilerParams(needs_layout_passes=False),
    scratch_types=[
        pltpu.VMEM((BUFLEN,), jnp.float32),
        pltpu.VMEM((BUFLEN,), jnp.float32),
        pltpu.VMEM((L * NBINS,), jnp.int32),
        pltpu.VMEM((NBINS,), jnp.int32),
        pltpu.SemaphoreType.DMA,
        pltpu.SemaphoreType.DMA,
    ],
)
def _hist_kernel(img_hbm, out_hbm, fbuf_a, fbuf_b, hist, obuf, sem_a, sem_b):
    wid = lax.axis_index("s") * NC + lax.axis_index("c")

    iota16 = lax.iota(jnp.int32, L)
    iota_u = lax.iota(jnp.uint32, L)
    ones = jnp.ones((L,), jnp.int32)
    mones = -ones
    zeros = jnp.zeros((L,), jnp.int32)
    mlast = iota16 < (L - 1)

    def zero_hist(i, carry):
        hist[pl.ds(i * L, L)] = zeros
        return carry

    lax.fori_loop(0, (L * NBINS) // L, zero_hist, 0)

    def dma_rows(bb, buf, sem):
        row0 = wid * ROWS_PER_W + bb * R
        copies = [
            pltpu.make_async_copy(img_hbm.at[row0 + r],
                                  buf.at[pl.ds(r * W, W)], sem)
            for r in range(R)
        ]
        halo = pltpu.make_async_copy(img_hbm.at[row0 + R],
                                     buf.at[pl.ds(R * W, W)], sem)
        return copies, halo, row0 + R < H

    def start_dma(bb, buf, sem):
        copies, halo, has_halo = dma_rows(bb, buf, sem)
        for c in copies:
            c.start()

        @pl.when(has_halo)
        def _():
            halo.start()

    def wait_dma(bb, buf, sem):
        copies, halo, has_halo = dma_rows(bb, buf, sem)
        for c in copies:
            c.wait()

        @pl.when(has_halo)
        def _():
            halo.wait()

    def process(bb, buf):
        row0 = wid * ROWS_PER_W + bb * R
        fast = row0 + R < H

        def conv_chunk(off):
            b = _to_bin(buf[pl.ds(off, L)])
            return plsc.bitcast((b << 4) | iota16, jnp.uint32)

        def scat(idx_u, val, mask=None):
            plsc.addupdate_scatter(hist, [plsc.bitcast(idx_u, jnp.int32)],
                                   val, mask=mask)

        @pl.when(fast)
        def _fast():
            def column(jbase, edge_mask):
                ia = conv_chunk(jbase)
                iar = conv_chunk(jbase + 1)
                for r in range(R):
                    down = (r + 1) * W
                    iad = conv_chunk(jbase + down)
                    iadr = conv_chunk(jbase + down + 1)
                    ix = jnp.maximum(ia, iad)
                    iy = jnp.maximum(ia, iar)
                    isq = jnp.maximum(iy, jnp.maximum(iad, iadr))
                    scat(ia, ones)
                    scat(ix, mones)
                    scat(iy, mones, mask=edge_mask)
                    scat(isq, ones, mask=edge_mask)
                    ia, iar = iad, iadr

            @plsc.parallel_loop(0, CHUNKS - 1)
            def chunkf(c):
                column(c * L, None)

            column((CHUNKS - 1) * L, mlast)

        @pl.when(jnp.logical_not(fast))
        def _slow():
            @plsc.parallel_loop(0, (R * W) // L, unroll=4)
            def conv(t):
                off = t * L
                buf[pl.ds(off, L)] = plsc.bitcast(conv_chunk(off),
                                                  jnp.float32)

            def bins(off):
                return plsc.bitcast(buf[pl.ds(off, L)], jnp.uint32)

            def retag(idx_u):
                return (idx_u & jnp.uint32(0xFFFFFFF0)) | iota_u

            def rowf(r, c0):
                gi = row0 + r

                @pl.when(gi < H - 1)
                def _full_row():
                    @plsc.parallel_loop(0, CHUNKS - 1, unroll=5)
                    def chunkf(c):
                        base = r * W + c * L
                        ia = bins(base)
                        iar = bins(base + 1)
                        iad = bins(base + W)
                        iadr = bins(base + W + 1)
                        ix = jnp.maximum(ia, iad)
                        iy = jnp.maximum(ia, iar)
                        isq = jnp.maximum(iy, jnp.maximum(iad, iadr))
                        scat(ia, ones)
                        scat(ix, mones)
                        scat(retag(iy), mones)
                        scat(retag(isq), ones)

                    base = r * W + (CHUNKS - 1) * L
                    ia = bins(base)
                    iar = bins(base + 1)
                    iad = bins(base + W)
                    iadr = bins(base + W + 1)
                    ix = jnp.maximum(ia, iad)
                    iy = jnp.maximum(ia, iar)
                    isq = jnp.maximum(iy, jnp.maximum(iad, iadr))
                    scat(ia, ones)
                    scat(ix, mones)
                    scat(retag(iy), mones, mask=mlast)
                    scat(retag(isq), ones, mask=mlast)

                @pl.when(gi == H - 1)
                def _last_row():
                    @plsc.parallel_loop(0, CHUNKS - 1, unroll=5)
                    def chunkv(c):
                        base = r * W + c * L
                        ia = bins(base)
                        iar = bins(base + 1)
                        scat(ia, ones)
                        scat(retag(jnp.maximum(ia, iar)), mones)

                    base = r * W + (CHUNKS - 1) * L
                    ia = bins(base)
                    iar = bins(base + 1)
                    scat(ia, ones)
                    scat(retag(jnp.maximum(ia, iar)), mones, mask=mlast)

                return c0

            lax.fori_loop(0, R, rowf, 0)

    start_dma(0, fbuf_a, sem_a)

    def outer(k, carry):
        b0 = 2 * k
        wait_dma(b0, fbuf_a, sem_a)
        start_dma(b0 + 1, fbuf_b, sem_b)
        process(b0, fbuf_a)
        wait_dma(b0 + 1, fbuf_b, sem_b)

        @pl.when(b0 + 2 < BLOCKS)
        def _():
            start_dma(b0 + 2, fbuf_a, sem_a)

        process(b0 + 1, fbuf_b)
        return carry

    lax.fori_loop(0, BLOCKS // 2, outer, 0)

    gidx = iota16 * L

    def fold(cidx, carry):
        acc = plsc.load_gather(hist, [gidx + cidx * (L * L)])
        for lane in range(1, L):
            acc = acc + plsc.load_gather(hist, [gidx + (cidx * (L * L) + lane)])
        obuf[pl.ds(cidx * L, L)] = acc
        return carry

    lax.fori_loop(0, NBINS // L, fold, 0)
    pltpu.sync_copy(obuf, out_hbm.at[wid])


@functools.partial(
    pl.kernel,
    out_type=jax.ShapeDtypeStruct((NBINS,), jnp.int32),
    mesh=_mesh,
    compiler_params=pltpu.CompilerParams(needs_layout_passes=False),
    scratch_types=[
        pltpu.VMEM((NW, NBINS), jnp.int32),
        pltpu.VMEM((NBINS,), jnp.int32),
    ],
)
def _finalize_kernel(part_hbm, out_hbm, pbuf, obuf):
    wid = lax.axis_index("s") * NC + lax.axis_index("c")

    onehot0 = (lax.iota(jnp.int32, L) == 0).astype(jnp.int32)
    fifteen = jnp.full((L,), L - 1, jnp.int32)

    @pl.when(wid == 0)
    def _():
        pltpu.sync_copy(part_hbm, pbuf)

        def chunk(cidx, carry_vec):
            off = cidx * L
            acc = pbuf[0, pl.ds(off, L)]
            for w in range(1, NW):
                acc = acc + pbuf[w, pl.ds(off, L)]
            acc = acc + carry_vec * onehot0
            cum = plsc.cumsum(acc)
            obuf[pl.ds(off, L)] = cum
            return cum.at[fifteen].get(mode="promise_in_bounds")

        lax.fori_loop(0, NBINS // L, chunk, jnp.zeros((L,), jnp.int32))
        pltpu.sync_copy(obuf, out_hbm)


def kernel(img_arr):
    part = _hist_kernel(img_arr)
    return _finalize_kernel(part)


# speedup vs baseline: 694.3307x; 1.2349x over previous
import functools

import jax
import jax.numpy as jnp
from jax import lax
from jax.experimental import pallas as pl
from jax.experimental.pallas import tpu as pltpu
from jax.experimental.pallas import tpu_sc as plsc

H = 4096
W = 4096
NBINS = 1024
NC = 2
NS = 16
NW = NC * NS
ROWS_PER_W = H // NW
R = 8
BLOCKS = ROWS_PER_W // R
L = 16
CHUNKS = W // L
BUFLEN = (R + 1) * W + L

_mesh = plsc.VectorSubcoreMesh(core_axis_name="c", subcore_axis_name="s")


def _to_bin(v):
    y = v * jnp.float32(NBINS - 1)
    ti = y.astype(jnp.int32)
    return jnp.where(ti.astype(jnp.float32) < y, ti + 1, ti)


@functools.partial(
    pl.kernel,
    out_type=jax.ShapeDtypeStruct((NW, NBINS), jnp.int32),
    mesh=_mesh,
    compiler_params=pltpu.CompilerParams(needs_layout_passes=False),
    scratch_types=[
        pltpu.VMEM((BUFLEN,), jnp.float32),
        pltpu.VMEM((BUFLEN,), jnp.float32),
        pltpu.VMEM((L * NBINS,), jnp.int32),
        pltpu.VMEM((NBINS,), jnp.int32),
        pltpu.SemaphoreType.DMA,
        pltpu.SemaphoreType.DMA,
    ],
)
def _hist_kernel(img_hbm, out_hbm, fbuf_a, fbuf_b, hist, obuf, sem_a, sem_b):
    wid = lax.axis_index("s") * NC + lax.axis_index("c")

    iota16 = lax.iota(jnp.int32, L)
    iota_u = lax.iota(jnp.uint32, L)
    ones = jnp.ones((L,), jnp.int32)
    mones = -ones
    zeros = jnp.zeros((L,), jnp.int32)
    mlast = iota16 < (L - 1)
    shift1 = jnp.minimum(iota16 + 1, L - 1)
    zero_idx = jnp.zeros((L,), jnp.int32)

    def zero_hist(i, carry):
        hist[pl.ds(i * L, L)] = zeros
        return carry

    lax.fori_loop(0, (L * NBINS) // L, zero_hist, 0)

    def dma_rows(bb, buf, sem):
        row0 = wid * ROWS_PER_W + bb * R
        copies = [
            pltpu.make_async_copy(img_hbm.at[row0 + r],
                                  buf.at[pl.ds(r * W, W)], sem)
            for r in range(R)
        ]
        halo = pltpu.make_async_copy(img_hbm.at[row0 + R],
                                     buf.at[pl.ds(R * W, W)], sem)
        return copies, halo, row0 + R < H

    def start_dma(bb, buf, sem):
        copies, halo, has_halo = dma_rows(bb, buf, sem)
        for c in copies:
            c.start()

        @pl.when(has_halo)
        def _():
            halo.start()

    def wait_dma(bb, buf, sem):
        copies, halo, has_halo = dma_rows(bb, buf, sem)
        for c in copies:
            c.wait()

        @pl.when(has_halo)
        def _():
            halo.wait()

    def process(bb, buf):
        row0 = wid * ROWS_PER_W + bb * R
        fast = row0 + R < H

        def conv_chunk(off):
            b = _to_bin(buf[pl.ds(off, L)])
            return plsc.bitcast((b << 4) | iota16, jnp.uint32)

        def scat(idx_u, val, mask=None):
            plsc.addupdate_scatter(hist, [plsc.bitcast(idx_u, jnp.int32)],
                                   val, mask=mask)

        def retag(idx_u):
            return (idx_u & jnp.uint32(0xFFFFFFF0)) | iota_u

        RH = R // 2

        def col_scatter(prev, sh, edge_mask):
            iy = [jnp.maximum(prev[i], sh[i]) for i in range(RH + 1)]
            for i in range(RH):
                ix = jnp.maximum(prev[i], prev[i + 1])
                isq = jnp.maximum(iy[i], iy[i + 1])
                scat(prev[i], ones)
                scat(ix, mones)
                scat(retag(iy[i]), mones, mask=edge_mask)
                scat(retag(isq), ones, mask=edge_mask)

        def walk(r0):
            def load_col(jb):
                return tuple(conv_chunk(jb + (r0 + i) * W)
                             for i in range(RH + 1))

            first = load_col(0)

            @plsc.parallel_loop(1, CHUNKS, carry=first)
            def chunkf(c, prev):
                cur = load_col(c * L)
                sh = [
                    jnp.where(
                        mlast,
                        prev[i].at[shift1].get(mode="promise_in_bounds"),
                        cur[i].at[zero_idx].get(mode="promise_in_bounds"),
                    )
                    for i in range(RH + 1)
                ]
                col_scatter(prev, sh, None)
                return cur

            last = chunkf
            sh = [last[i].at[shift1].get(mode="promise_in_bounds")
                  for i in range(RH + 1)]
            col_scatter(last, sh, mlast)

        @pl.when(fast)
        def _fast():
            walk(0)
            walk(RH)

        @pl.when(jnp.logical_not(fast))
        def _slow():
            @plsc.parallel_loop(0, (R * W) // L, unroll=4)
            def conv(t):
                off = t * L
                buf[pl.ds(off, L)] = plsc.bitcast(conv_chunk(off),
                                                  jnp.float32)

            def bins(off):
                return plsc.bitcast(buf[pl.ds(off, L)], jnp.uint32)

            def rowf(r, c0):
                gi = row0 + r

                @pl.when(gi < H - 1)
                def _full_row():
                    @plsc.parallel_loop(0, CHUNKS - 1, unroll=5)
                    def chunkf(c):
                        base = r * W + c * L
                        ia = bins(base)
                        iar = bins(base + 1)
                        iad = bins(base + W)
                        iadr = bins(base + W + 1)
                        ix = jnp.maximum(ia, iad)
                        iy = jnp.maximum(ia, iar)
                        isq = jnp.maximum(iy, jnp.maximum(iad, iadr))
                        scat(ia, ones)
                        scat(ix, mones)
                        scat(retag(iy), mones)
                        scat(retag(isq), ones)

                    base = r * W + (CHUNKS - 1) * L
                    ia = bins(base)
                    iar = bins(base + 1)
                    iad = bins(base + W)
                    iadr = bins(base + W + 1)
                    ix = jnp.maximum(ia, iad)
                    iy = jnp.maximum(ia, iar)
                    isq = jnp.maximum(iy, jnp.maximum(iad, iadr))
                    scat(ia, ones)
                    scat(ix, mones)
                    scat(retag(iy), mones, mask=mlast)
                    scat(retag(isq), ones, mask=mlast)

                @pl.when(gi == H - 1)
                def _last_row():
                    @plsc.parallel_loop(0, CHUNKS - 1, unroll=5)
                    def chunkv(c):
                        base = r * W + c * L
                        ia = bins(base)
                        iar = bins(base + 1)
                        scat(ia, ones)
                        scat(retag(jnp.maximum(ia, iar)), mones)

                    base = r * W + (CHUNKS - 1) * L
                    ia = bins(base)
                    iar = bins(base + 1)
                    scat(ia, ones)
                    scat(retag(jnp.maximum(ia, iar)), mones, mask=mlast)

                return c0

            lax.fori_loop(0, R, rowf, 0)

    start_dma(0, fbuf_a, sem_a)

    def outer(k, carry):
        b0 = 2 * k
        wait_dma(b0, fbuf_a, sem_a)
        start_dma(b0 + 1, fbuf_b, sem_b)
        process(b0, fbuf_a)
        wait_dma(b0 + 1, fbuf_b, sem_b)

        @pl.when(b0 + 2 < BLOCKS)
        def _():
            start_dma(b0 + 2, fbuf_a, sem_a)

        process(b0 + 1, fbuf_b)
        return carry

    lax.fori_loop(0, BLOCKS // 2, outer, 0)

    gidx = iota16 * L

    def fold(cidx, carry):
        acc = plsc.load_gather(hist, [gidx + cidx * (L * L)])
        for lane in range(1, L):
            acc = acc + plsc.load_gather(hist, [gidx + (cidx * (L * L) + lane)])
        obuf[pl.ds(cidx * L, L)] = acc
        return carry

    lax.fori_loop(0, NBINS // L, fold, 0)
    pltpu.sync_copy(obuf, out_hbm.at[wid])


@functools.partial(
    pl.kernel,
    out_type=jax.ShapeDtypeStruct((NBINS,), jnp.int32),
    mesh=_mesh,
    compiler_params=pltpu.CompilerParams(needs_layout_passes=False),
    scratch_types=[
        pltpu.VMEM((NW, NBINS), jnp.int32),
        pltpu.VMEM((NBINS,), jnp.int32),
    ],
)
def _finalize_kernel(part_hbm, out_hbm, pbuf, obuf):
    wid = lax.axis_index("s") * NC + lax.axis_index("c")

    onehot0 = (lax.iota(jnp.int32, L) == 0).astype(jnp.int32)
    fifteen = jnp.full((L,), L - 1, jnp.int32)

    @pl.when(wid == 0)
    def _():
        pltpu.sync_copy(part_hbm, pbuf)

        def chunk(cidx, carry_vec):
            off = cidx * L
            acc = pbuf[0, pl.ds(off, L)]
            for w in range(1, NW):
                acc = acc + pbuf[w, pl.ds(off, L)]
            acc = acc + carry_vec * onehot0
            cum = plsc.cumsum(acc)
            obuf[pl.ds(off, L)] = cum
            return cum.at[fifteen].get(mode="promise_in_bounds")

        lax.fori_loop(0, NBINS // L, chunk, jnp.zeros((L,), jnp.int32))
        pltpu.sync_copy(obuf, out_hbm)


def kernel(img_arr):
    part = _hist_kernel(img_arr)
    return _finalize_kernel(part)


# speedup vs baseline: 705.7688x; 1.0165x over previous
import functools

import jax
import jax.numpy as jnp
from jax import lax
from jax.experimental import pallas as pl
from jax.experimental.pallas import tpu as pltpu
from jax.experimental.pallas import tpu_sc as plsc

H = 4096
W = 4096
NBINS = 1024
NC = 2
NS = 16
NW = NC * NS
ROWS_PER_W = H // NW
R = 8
BLOCKS = ROWS_PER_W // R
L = 16
CHUNKS = W // L
BUFLEN = (R + 1) * W + L

_mesh = plsc.VectorSubcoreMesh(core_axis_name="c", subcore_axis_name="s")


def _to_bin(v):
    y = v * jnp.float32(NBINS - 1)
    ti = y.astype(jnp.int32)
    return jnp.where(ti.astype(jnp.float32) < y, ti + 1, ti)


@functools.partial(
    pl.kernel,
    out_type=jax.ShapeDtypeStruct((NW, NBINS), jnp.int32),
    mesh=_mesh,
    compiler_params=pltpu.CompilerParams(needs_layout_passes=False),
    scratch_types=[
        pltpu.VMEM((BUFLEN,), jnp.float32),
        pltpu.VMEM((BUFLEN,), jnp.float32),
        pltpu.VMEM((L * NBINS,), jnp.int32),
        pltpu.VMEM((NBINS,), jnp.int32),
        pltpu.SemaphoreType.DMA,
        pltpu.SemaphoreType.DMA,
    ],
)
def _hist_kernel(img_hbm, out_hbm, fbuf_a, fbuf_b, hist, obuf, sem_a, sem_b):
    wid = lax.axis_index("s") * NC + lax.axis_index("c")

    iota16 = lax.iota(jnp.int32, L)
    iota_u = lax.iota(jnp.uint32, L)
    ones = jnp.ones((L,), jnp.int32)
    mones = -ones
    zeros = jnp.zeros((L,), jnp.int32)
    mlast = iota16 < (L - 1)
    shift1 = jnp.minimum(iota16 + 1, L - 1)
    zero_idx = jnp.zeros((L,), jnp.int32)


    def dma_rows(bb, buf, sem):
        row0 = wid * ROWS_PER_W + bb * R
        copies = [
            pltpu.make_async_copy(img_hbm.at[row0 + r],
                                  buf.at[pl.ds(r * W, W)], sem)
            for r in range(R)
        ]
        halo = pltpu.make_async_copy(img_hbm.at[row0 + R],
                                     buf.at[pl.ds(R * W, W)], sem)
        return copies, halo, row0 + R < H

    def start_dma(bb, buf, sem):
        copies, halo, has_halo = dma_rows(bb, buf, sem)
        for c in copies:
            c.start()

        @pl.when(has_halo)
        def _():
            halo.start()

    def wait_dma(bb, buf, sem):
        copies, halo, has_halo = dma_rows(bb, buf, sem)
        for c in copies:
            c.wait()

        @pl.when(has_halo)
        def _():
            halo.wait()

    def process(bb, buf):
        row0 = wid * ROWS_PER_W + bb * R
        fast = row0 + R < H

        def conv_chunk(off):
            b = _to_bin(buf[pl.ds(off, L)])
            return plsc.bitcast((b << 4) | iota16, jnp.uint32)

        def scat(idx_u, val, mask=None):
            plsc.addupdate_scatter(hist, [plsc.bitcast(idx_u, jnp.int32)],
                                   val, mask=mask)

        def retag(idx_u):
            return (idx_u & jnp.uint32(0xFFFFFFF0)) | iota_u

        RH = R // 2

        def col_scatter(prev, sh, edge_mask):
            iy = [jnp.maximum(prev[i], sh[i]) for i in range(RH + 1)]
            for i in range(RH):
                ix = jnp.maximum(prev[i], prev[i + 1])
                isq = jnp.maximum(iy[i], iy[i + 1])
                scat(prev[i], ones)
                scat(ix, mones)
                scat(retag(iy[i]), mones, mask=edge_mask)
                scat(retag(isq), ones, mask=edge_mask)

        def walk(r0):
            def load_col(jb):
                return tuple(conv_chunk(jb + (r0 + i) * W)
                             for i in range(RH + 1))

            first = load_col(0)

            @plsc.parallel_loop(1, CHUNKS, carry=first)
            def chunkf(c, prev):
                cur = load_col(c * L)
                sh = [
                    jnp.where(
                        mlast,
                        prev[i].at[shift1].get(mode="promise_in_bounds"),
                        cur[i].at[zero_idx].get(mode="promise_in_bounds"),
                    )
                    for i in range(RH + 1)
                ]
                col_scatter(prev, sh, None)
                return cur

            last = chunkf
            sh = [last[i].at[shift1].get(mode="promise_in_bounds")
                  for i in range(RH + 1)]
            col_scatter(last, sh, mlast)

        @pl.when(fast)
        def _fast():
            walk(0)
            walk(RH)

        @pl.when(jnp.logical_not(fast))
        def _slow():
            @plsc.parallel_loop(0, (R * W) // L, unroll=4)
            def conv(t):
                off = t * L
                buf[pl.ds(off, L)] = plsc.bitcast(conv_chunk(off),
                                                  jnp.float32)

            def bins(off):
                return plsc.bitcast(buf[pl.ds(off, L)], jnp.uint32)

            def rowf(r, c0):
                gi = row0 + r

                @pl.when(gi < H - 1)
                def _full_row():
                    @plsc.parallel_loop(0, CHUNKS - 1, unroll=5)
                    def chunkf(c):
                        base = r * W + c * L
                        ia = bins(base)
                        iar = bins(base + 1)
                        iad = bins(base + W)
                        iadr = bins(base + W + 1)
                        ix = jnp.maximum(ia, iad)
                        iy = jnp.maximum(ia, iar)
                        isq = jnp.maximum(iy, jnp.maximum(iad, iadr))
                        scat(ia, ones)
                        scat(ix, mones)
                        scat(retag(iy), mones)
                        scat(retag(isq), ones)

                    base = r * W + (CHUNKS - 1) * L
                    ia = bins(base)
                    iar = bins(base + 1)
                    iad = bins(base + W)
                    iadr = bins(base + W + 1)
                    ix = jnp.maximum(ia, iad)
                    iy = jnp.maximum(ia, iar)
                    isq = jnp.maximum(iy, jnp.maximum(iad, iadr))
                    scat(ia, ones)
                    scat(ix, mones)
                    scat(retag(iy), mones, mask=mlast)
                    scat(retag(isq), ones, mask=mlast)

                @pl.when(gi == H - 1)
                def _last_row():
                    @plsc.parallel_loop(0, CHUNKS - 1, unroll=5)
                    def chunkv(c):
                        base = r * W + c * L
                        ia = bins(base)
                        iar = bins(base + 1)
                        scat(ia, ones)
                        scat(retag(jnp.maximum(ia, iar)), mones)

                    base = r * W + (CHUNKS - 1) * L
                    ia = bins(base)
                    iar = bins(base + 1)
                    scat(ia, ones)
                    scat(retag(jnp.maximum(ia, iar)), mones, mask=mlast)

                return c0

            lax.fori_loop(0, R, rowf, 0)

    start_dma(0, fbuf_a, sem_a)

    def zero_hist(i, carry):
        hist[pl.ds(i * L, L)] = zeros
        return carry

    lax.fori_loop(0, (L * NBINS) // L, zero_hist, 0)

    def outer(k, carry):
        b0 = 2 * k
        wait_dma(b0, fbuf_a, sem_a)
        start_dma(b0 + 1, fbuf_b, sem_b)
        process(b0, fbuf_a)
        wait_dma(b0 + 1, fbuf_b, sem_b)

        @pl.when(b0 + 2 < BLOCKS)
        def _():
            start_dma(b0 + 2, fbuf_a, sem_a)

        process(b0 + 1, fbuf_b)
        return carry

    lax.fori_loop(0, BLOCKS // 2, outer, 0)

    gidx = iota16 * L

    def fold(cidx, carry):
        acc = plsc.load_gather(hist, [gidx + cidx * (L * L)])
        for lane in range(1, L):
            acc = acc + plsc.load_gather(hist, [gidx + (cidx * (L * L) + lane)])
        obuf[pl.ds(cidx * L, L)] = acc
        return carry

    lax.fori_loop(0, NBINS // L, fold, 0)
    pltpu.sync_copy(obuf, out_hbm.at[wid])


@functools.partial(
    pl.kernel,
    out_type=jax.ShapeDtypeStruct((NBINS,), jnp.int32),
    mesh=_mesh,
    compiler_params=pltpu.CompilerParams(needs_layout_passes=False),
    scratch_types=[
        pltpu.VMEM((NW, NBINS), jnp.int32),
        pltpu.VMEM((NBINS,), jnp.int32),
    ],
)
def _finalize_kernel(part_hbm, out_hbm, pbuf, obuf):
    wid = lax.axis_index("s") * NC + lax.axis_index("c")

    onehot0 = (lax.iota(jnp.int32, L) == 0).astype(jnp.int32)
    fifteen = jnp.full((L,), L - 1, jnp.int32)

    @pl.when(wid == 0)
    def _():
        pltpu.sync_copy(part_hbm, pbuf)

        def chunk(cidx, carry_vec):
            off = cidx * L
            acc = pbuf[0, pl.ds(off, L)]
            for w in range(1, NW):
                acc = acc + pbuf[w, pl.ds(off, L)]
            acc = acc + carry_vec * onehot0
            cum = plsc.cumsum(acc)
            obuf[pl.ds(off, L)] = cum
            return cum.at[fifteen].get(mode="promise_in_bounds")

        lax.fori_loop(0, NBINS // L, chunk, jnp.zeros((L,), jnp.int32))
        pltpu.sync_copy(obuf, out_hbm)


def kernel(img_arr):
    part = _hist_kernel(img_arr)
    return _finalize_kernel(part)
